# baseline (speedup 1.0000x reference)
import functools

import jax
import jax.numpy as jnp
from jax import lax
from jax.experimental import pallas as pl
from jax.experimental.pallas import tpu as pltpu
from jax.experimental.pallas import tpu_sc as plsc

N = 10000
E = 320000
G = 100
D = 128
NHEADS = 4
ALPHA = 0.5
R = 25
NCLS = 10
M = 160

NC, NS = 2, 16
NW = NC * NS
CH = 128
NPAD = 10240
DUMP = 10220
CNT_BASE = 10000
DUMP_CNT = 10230

_MESH = dict(core_axis_name="c", subcore_axis_name="s", num_cores=NC,
             num_subcores=NS)


def _ceil_div(a, b):
    return -(-a // b)


@functools.partial(jax.jit, static_argnames=("d", "k"))
def _sc_prop(table, src3, dst3, zeros, *, d, k):
    rows_pt = NPAD // NS
    mesh = plsc.VectorSubcoreMesh(**_MESH)

    @functools.partial(
        pl.kernel,
        out_type=jax.ShapeDtypeStruct((NC, NPAD, d), jnp.float32),
        mesh=mesh,
        scratch_types=[
            pltpu.VMEM((k, CH), jnp.int32),
            pltpu.VMEM((k, CH), jnp.int32),
            pltpu.VMEM((CH, d), jnp.float32),
            pltpu.VMEM_SHARED((NPAD, d), jnp.float32),
            pltpu.SemaphoreType.DMA,
        ],
    )
    def kfn(table_hbm, src_hbm, dst_hbm, zero_hbm, out_hbm,
            sidx, didx, buf, accum, sem):
        c = lax.axis_index("c")
        s = lax.axis_index("s")
        w = c * NS + s
        pltpu.sync_copy(zero_hbm, accum.at[pl.ds(s * rows_pt, rows_pt)])
        pltpu.sync_copy(src_hbm.at[w], sidx)
        pltpu.sync_copy(dst_hbm.at[w], didx)
        plsc.subcore_barrier()

        def body(j, carry):
            pltpu.async_copy(table_hbm.at[sidx.at[j]], buf, sem).wait()
            pltpu.sync_copy(buf, accum.at[didx.at[j]], add=True)
            return carry

        lax.fori_loop(0, k, body, 0)
        plsc.subcore_barrier()
        pltpu.sync_copy(accum.at[pl.ds(s * rows_pt, rows_pt)],
                        out_hbm.at[c, pl.ds(s * rows_pt, rows_pt)])

    return kfn(table, src3, dst3, zeros)


@functools.partial(jax.jit, static_argnames=("k",))
def _sc_counts(dst3, ones, zeros, *, k):
    rows_pt = NPAD // NS
    mesh = plsc.VectorSubcoreMesh(**_MESH)

    @functools.partial(
        pl.kernel,
        out_type=jax.ShapeDtypeStruct((NC, NPAD, 16), jnp.float32),
        mesh=mesh,
        scratch_types=[
            pltpu.VMEM((k, CH), jnp.int32),
            pltpu.VMEM((CH, 16), jnp.float32),
            pltpu.VMEM_SHARED((NPAD, 16), jnp.float32),
        ],
        compiler_params=pltpu.CompilerParams(use_tc_tiling_on_sc=False),
    )
    def kfn(dst_hbm, ones_hbm, zero_hbm, out_hbm, didx, buf, accum):
        c = lax.axis_index("c")
        s = lax.axis_index("s")
        w = c * NS + s
        pltpu.sync_copy(zero_hbm, accum.at[pl.ds(s * rows_pt, rows_pt)])
        pltpu.sync_copy(dst_hbm.at[w], didx)
        pltpu.sync_copy(ones_hbm, buf)
        plsc.subcore_barrier()

        def body(j, carry):
            pltpu.sync_copy(buf, accum.at[didx.at[j]], add=True)
            return carry

        lax.fori_loop(0, k, body, 0)
        plsc.subcore_barrier()
        pltpu.sync_copy(accum.at[pl.ds(s * rows_pt, rows_pt)],
                        out_hbm.at[c, pl.ds(s * rows_pt, rows_pt)])

    return kfn(dst3, ones, zeros)


@functools.partial(jax.jit, static_argnames=("k",))
def _sc_prop_narrow(t, src3, dst3, zeros, *, k):
    rows_pt = NPAD // NS
    mesh = plsc.VectorSubcoreMesh(**_MESH)

    @functools.partial(
        pl.kernel,
        out_type=jax.ShapeDtypeStruct((NC, NPAD, 16), jnp.float32),
        mesh=mesh,
        scratch_types=[
            pltpu.VMEM((N,), jnp.float32),
            pltpu.VMEM((k, CH), jnp.int32),
            pltpu.VMEM((k, CH), jnp.int32),
            pltpu.VMEM((CH, 16), jnp.float32),
            pltpu.VMEM_SHARED((NPAD, 16), jnp.float32),
        ],
        compiler_params=pltpu.CompilerParams(
            needs_layout_passes=False, use_tc_tiling_on_sc=False),
    )
    def kfn(t_hbm, src_hbm, dst_hbm, zero_hbm, out_hbm,
            tv, sidx, didx, buf, accum):
        c = lax.axis_index("c")
        s = lax.axis_index("s")
        w = c * NS + s
        pltpu.sync_copy(zero_hbm, accum.at[pl.ds(s * rows_pt, rows_pt)])
        pltpu.sync_copy(t_hbm, tv)
        pltpu.sync_copy(src_hbm.at[w], sidx)
        pltpu.sync_copy(dst_hbm.at[w], didx)
        pltpu.sync_copy(zero_hbm.at[pl.ds(0, CH)], buf)
        plsc.subcore_barrier()
        lane = lax.iota(jnp.int32, 16)
        col0 = jnp.zeros((16,), jnp.int32)

        def chunk(j, carry):
            def grp(g, c2):
                sv = sidx[j, pl.ds(g * 16, 16)]
                vals = plsc.load_gather(tv, [sv])
                plsc.store_scatter(buf, [g * 16 + lane, col0], vals)
                return c2

            lax.fori_loop(0, 8, grp, 0)
            pltpu.sync_copy(buf, accum.at[didx.at[j]], add=True)
            return carry

        lax.fori_loop(0, k, chunk, 0)
        plsc.subcore_barrier()
        pltpu.sync_copy(accum.at[pl.ds(s * rows_pt, rows_pt)],
                        out_hbm.at[c, pl.ds(s * rows_pt, rows_pt)])

    return kfn(t, src3, dst3, zeros)


@functools.partial(jax.jit, static_argnames=("k",))
def _sc_gather_narrow(t, idx3, *, k):
    mesh = plsc.VectorSubcoreMesh(**_MESH)

    @functools.partial(
        pl.kernel,
        out_type=jax.ShapeDtypeStruct((NW * k * CH,), jnp.float32),
        mesh=mesh,
        scratch_types=[
            pltpu.VMEM((N,), jnp.float32),
            pltpu.VMEM((k, CH), jnp.int32),
            pltpu.VMEM((k * CH,), jnp.float32),
        ],
        compiler_params=pltpu.CompilerParams(needs_layout_passes=False),
    )
    def kfn(t_hbm, idx_hbm, out_hbm, tv, idxb, obuf):
        c = lax.axis_index("c")
        s = lax.axis_index("s")
        w = c * NS + s
        pltpu.sync_copy(t_hbm, tv)
        pltpu.sync_copy(idx_hbm.at[w], idxb)

        def chunk(j, carry):
            def grp(g, c2):
                sv = idxb[j, pl.ds(g * 16, 16)]
                obuf[pl.ds(j * CH + g * 16, 16)] = plsc.load_gather(tv, [sv])
                return c2

            lax.fori_loop(0, 8, grp, 0)
            return carry

        lax.fori_loop(0, k, chunk, 0)
        pltpu.sync_copy(obuf, out_hbm.at[pl.ds(w * k * CH, k * CH)])

    return kfn(t, idx3)


@functools.partial(jax.jit, static_argnames=("d", "k"))
def _sc_gather(table, idx3, *, d, k):
    mesh = plsc.VectorSubcoreMesh(**_MESH)

    @functools.partial(
        pl.kernel,
        out_type=jax.ShapeDtypeStruct((NW * k * CH, d), jnp.float32),
        mesh=mesh,
        scratch_types=[
            pltpu.VMEM((k, CH), jnp.int32),
            pltpu.VMEM((CH, d), jnp.float32),
            pltpu.SemaphoreType.DMA,
        ],
    )
    def kfn(table_hbm, idx_hbm, out_hbm, idxb, buf, sem):
        c = lax.axis_index("c")
        s = lax.axis_index("s")
        w = c * NS + s
        pltpu.sync_copy(idx_hbm.at[w], idxb)

        def body(j, carry):
            pltpu.async_copy(table_hbm.at[idxb.at[j]], buf, sem).wait()
            pltpu.sync_copy(buf, out_hbm.at[pl.ds(w * k * CH + j * CH, CH)])
            return carry

        lax.fori_loop(0, k, body, 0)

    return kfn(table, idx3)


def _pad_to_tiles(v, fill, k):
    tot = NW * k * CH
    v = jnp.concatenate(
        [v.astype(jnp.int32),
         jnp.full((tot - v.shape[0],), fill, jnp.int32)])
    return v.reshape(NW, k, CH)


BR = 1000
GN = N // BR

_rows = lambda i: (i, 0)
_rows3 = lambda i: (0, i, 0)
_full = lambda i: (0, 0)
_full1 = lambda i: (0,)


def _dinv_of(cnt_ref):
    deg = cnt_ref[0][:, 0:1] + cnt_ref[1][:, 0:1] + 1.0
    return lax.rsqrt(jnp.maximum(deg, 1e-12))


@jax.jit
def _tc_prep(x, cnt, We, be, Wa, Wb):
    def body(x_ref, cnt_ref, we_ref, be_ref, wa_ref, wb_ref, ta_ref, tb_ref):
        dinv = _dinv_of(cnt_ref)
        h = jnp.dot(x_ref[...], we_ref[...]) + be_ref[...]
        ta_ref[...] = dinv * jnp.dot(h, wa_ref[...])
        tb_ref[...] = dinv * jnp.dot(h, wb_ref[...])

    return pl.pallas_call(
        body, grid=(GN,),
        in_specs=[pl.BlockSpec((BR, D), _rows),
                  pl.BlockSpec((NC, BR, 16), _rows3),
                  pl.BlockSpec((D, D), _full),
                  pl.BlockSpec((D,), _full1),
                  pl.BlockSpec((D, D), _full),
                  pl.BlockSpec((D, D), _full)],
        out_specs=[pl.BlockSpec((BR, D), _rows)] * 2,
        out_shape=[jax.ShapeDtypeStruct((N, D), jnp.float32)] * 2,
    )(x, cnt, We, be, Wa, Wb)


@functools.partial(jax.jit, static_argnames=("has_next",))
def _tc_layer(za, zb, tsa, tsb, cnt, ba, bb, Wna, Wnb, Wws2, Wws1, bws1,
              *, has_next):
    def body(za_ref, zb_ref, tsa_ref, tsb_ref, cnt_ref, ba_ref, bb_ref,
             wna_ref, wnb_ref, wws2_ref, wws1_ref, bws1_ref, *out_refs):
        if has_next:
            x_ref, tna_ref, tnb_ref, ts_ref, sa_ref = out_refs
        else:
            x_ref, ts_ref, sa_ref = out_refs
        dinv = _dinv_of(cnt_ref)
        conv_a = dinv * (za_ref[0] + za_ref[1] + tsa_ref[...]) + ba_ref[...]
        conv_b = dinv * (zb_ref[0] + zb_ref[1] + tsb_ref[...]) + bb_ref[...]
        xi = jax.nn.relu(conv_a) + jax.nn.relu(conv_b)
        x_ref[...] = xi
        if has_next:
            tna_ref[...] = dinv * jnp.dot(xi, wna_ref[...])
            tnb_ref[...] = dinv * jnp.dot(xi, wnb_ref[...])
        ts_ref[...] = dinv * jnp.dot(xi, wws2_ref[...])
        sa_ref[...] = jnp.dot(xi, wws1_ref[...]) + bws1_ref[...]

    n_out = 5 if has_next else 3
    shapes = [jax.ShapeDtypeStruct((N, D), jnp.float32)] * (3 if has_next else 1) \
        + [jax.ShapeDtypeStruct((N, 1), jnp.float32)] * 2
    specs = [pl.BlockSpec((BR, D), _rows)] * (3 if has_next else 1) \
        + [pl.BlockSpec((BR, 1), _rows)] * 2
    assert len(shapes) == n_out
    return pl.pallas_call(
        body, grid=(GN,),
        in_specs=[pl.BlockSpec((NC, BR, D), _rows3),
                  pl.BlockSpec((NC, BR, D), _rows3),
                  pl.BlockSpec((BR, D), _rows),
                  pl.BlockSpec((BR, D), _rows),
                  pl.BlockSpec((NC, BR, 16), _rows3),
                  pl.BlockSpec((D,), _full1),
                  pl.BlockSpec((D,), _full1),
                  pl.BlockSpec((D, D), _full),
                  pl.BlockSpec((D, D), _full),
                  pl.BlockSpec((D, 1), _full),
                  pl.BlockSpec((D, 1), _full),
                  pl.BlockSpec((1,), _full1)],
        out_specs=specs,
        out_shape=shapes,
    )(za, zb, tsa, tsb, cnt, ba, bb, Wna, Wnb, Wws2, Wws1, bws1)


@jax.jit
def _tc_combine(x1, x2, x3, sa1, sa2, sa3, ts1, ts2, ts3, zt1, zt2, zt3,
                cnt, bws2, Wk, Wv, Wps2, Wps1, bps1):
    def body(x1_ref, x2_ref, x3_ref, sa1_ref, sa2_ref, sa3_ref,
             ts1_ref, ts2_ref, ts3_ref, zt1_ref, zt2_ref, zt3_ref,
             cnt_ref, bws2_ref, wk_ref, wv_ref, wps2_ref, wps1_ref, bps1_ref,
             xm_ref, tk_ref, tv_ref, tps_ref, sap_ref):
        dinv = _dinv_of(cnt_ref)

        def wcol(sa_ref, ts_ref, zt_ref):
            convn = dinv * (zt_ref[0][:, 0:1] + zt_ref[1][:, 0:1]
                            + ts_ref[...]) + bws2_ref[...]
            return ALPHA * sa_ref[...] + (1 - ALPHA) * convn

        wcat = jnp.concatenate(
            [wcol(sa1_ref, ts1_ref, zt1_ref),
             wcol(sa2_ref, ts2_ref, zt2_ref),
             wcol(sa3_ref, ts3_ref, zt3_ref)], axis=1)
        wsm = jax.nn.softmax(wcat, axis=-1)
        xm = (wsm[:, 0:1] * x1_ref[...] + wsm[:, 1:2] * x2_ref[...]
              + wsm[:, 2:3] * x3_ref[...])
        xm_ref[...] = xm
        tk_ref[...] = dinv * jnp.dot(xm, wk_ref[...])
        tv_ref[...] = dinv * jnp.dot(xm, wv_ref[...])
        tps_ref[...] = dinv * jnp.dot(xm, wps2_ref[...])
        sap_ref[...] = jnp.dot(xm, wps1_ref[...]) + bps1_ref[...]

    return pl.pallas_call(
        body, grid=(GN,),
        in_specs=[pl.BlockSpec((BR, D), _rows)] * 3
        + [pl.BlockSpec((BR, 1), _rows)] * 6
        + [pl.BlockSpec((NC, BR, 16), _rows3)] * 4
        + [pl.BlockSpec((1,), _full1),
           pl.BlockSpec((D, D), _full), pl.BlockSpec((D, D), _full),
           pl.BlockSpec((D, 1), _full), pl.BlockSpec((D, 1), _full),
           pl.BlockSpec((1,), _full1)],
        out_specs=[pl.BlockSpec((BR, D), _rows)] * 3
        + [pl.BlockSpec((BR, 1), _rows)] * 2,
        out_shape=[jax.ShapeDtypeStruct((N, D), jnp.float32)] * 3
        + [jax.ShapeDtypeStruct((N, 1), jnp.float32)] * 2,
    )(x1, x2, x3, sa1, sa2, sa3, ts1, ts2, ts3, zt1, zt2, zt3,
      cnt, bws2, Wk, Wv, Wps2, Wps1, bps1)


@jax.jit
def _tc_pool(zk, zv, zp, tk, tv, tps, sap, cnt, bk, bv, bps2):
    def body(zk_ref, zv_ref, zp_ref, tk_ref, tv_ref, tps_ref, sap_ref,
             cnt_ref, bk_ref, bv_ref, bps2_ref, kv_ref, s_ref):
        dinv = _dinv_of(cnt_ref)
        kv_ref[:, 0:D] = dinv * (zk_ref[0] + zk_ref[1] + tk_ref[...]) \
            + bk_ref[...]
        kv_ref[:, D:2 * D] = dinv * (zv_ref[0] + zv_ref[1] + tv_ref[...]) \
            + bv_ref[...]
        convp = dinv * (zp_ref[0][:, 0:1] + zp_ref[1][:, 0:1]
                        + tps_ref[...]) + bps2_ref[...]
        s_ref[...] = ALPHA * sap_ref[...] + (1 - ALPHA) * convp

    return pl.pallas_call(
        body, grid=(GN,),
        in_specs=[pl.BlockSpec((NC, BR, D), _rows3)] * 2
        + [pl.BlockSpec((NC, BR, 16), _rows3)]
        + [pl.BlockSpec((BR, D), _rows)] * 2
        + [pl.BlockSpec((BR, 1), _rows)] * 2
        + [pl.BlockSpec((NC, BR, 16), _rows3)]
        + [pl.BlockSpec((D,), _full1)] * 2
        + [pl.BlockSpec((1,), _full1)],
        out_specs=[pl.BlockSpec((BR, 2 * D), _rows),
                   pl.BlockSpec((BR, 1), _rows)],
        out_shape=[jax.ShapeDtypeStruct((N, 2 * D), jnp.float32),
                   jax.ShapeDtypeStruct((N, 1), jnp.float32)],
    )(zk, zv, zp, tk, tv, tps, sap, cnt, bk, bv, bps2)


@jax.jit
def _tc_batchpos(cnt):
    def body(cnt_ref, didx_ref, mask_ref):
        counts = (cnt_ref[0][0:G, 0:1] + cnt_ref[1][0:G, 0:1])
        row = lax.broadcasted_iota(jnp.int32, (G, G), 0)
        col = lax.broadcasted_iota(jnp.int32, (G, G), 1)
        tri = (col < row).astype(jnp.float32)
        starts = jax.lax.dot_general(
            tri, counts, (((1,), (0,)), ((), ())),
            precision=jax.lax.Precision.HIGHEST)
        midx = lax.broadcasted_iota(jnp.int32, (G, M), 1)
        didx = jnp.clip(starts.astype(jnp.int32) + midx, 0, N - 1)
        didx_ref[...] = didx
        mask_ref[...] = (midx < counts.astype(jnp.int32)).astype(jnp.float32)

    return pl.pallas_call(
        body, grid=(1,),
        in_specs=[pl.BlockSpec((NC, 200, 16),
                               lambda i: (0, CNT_BASE // 200, 0))],
        out_specs=[pl.BlockSpec((G, M), _rows)] * 2,
        out_shape=[jax.ShapeDtypeStruct((G, M), jnp.int32),
                   jax.ShapeDtypeStruct((G, M), jnp.float32)],
    )(cnt)


@jax.jit
def _tc_topk(sd, maskf, didx):
    def body(sd_ref, mask_ref, didx_ref, vals_ref, sel_ref):
        occ = mask_ref[...] > 0.0
        cur = jnp.where(occ, sd_ref[...], -1e30)
        di = jnp.where(occ, didx_ref[...], 0)
        iot = lax.broadcasted_iota(jnp.int32, (G, M), 1)
        big = jnp.int32(1 << 30)
        for r in range(R):
            mx = jnp.max(cur, axis=1, keepdims=True)
            ismax = cur == mx
            am = jnp.min(jnp.where(ismax, iot, big), axis=1, keepdims=True)
            take = iot == am
            vals_ref[:, r:r + 1] = mx
            sel_ref[:, r:r + 1] = jnp.max(jnp.where(take, di, 0), axis=1,
                                          keepdims=True)
            cur = jnp.where(take, jnp.float32(-3e38), cur)

    return pl.pallas_call(
        body, grid=(1,),
        in_specs=[pl.BlockSpec((G, M), _rows)] * 3,
        out_specs=[pl.BlockSpec((G, R), _rows)] * 2,
        out_shape=[jax.ShapeDtypeStruct((G, R), jnp.float32),
                   jax.ShapeDtypeStruct((G, R), jnp.int32)],
    )(sd, maskf, didx)


@jax.jit
def _tc_attn(xrows, vals, kvd, mask_m1, mask_1m, Wq, bq, Wo, bo,
             wread, bread, Wl1, bl1, Wl2, bl2):
    dh = D // NHEADS

    def body(xr_ref, vals_ref, kvd_ref, mm1_ref, m1m_ref, wq_ref, bq_ref,
             wo_ref, bo_ref, wr_ref, br_ref, wl1_ref, bl1_ref,
             wl2_ref, bl2_ref, logp_ref, gv_ref):
        v = vals_ref[0]
        ok = v > -1e29
        xp = jnp.where(ok, xr_ref[0] * jnp.tanh(v), 0.0)
        Q = jnp.dot(xp, wq_ref[...]) + bq_ref[...]
        mcol = mm1_ref[0]
        occ = m1m_ref[0] > 0.0
        kd = kvd_ref[0][:, 0:D] * mcol
        vd = kvd_ref[0][:, D:2 * D] * mcol
        outs = []
        scale = 1.0 / jnp.sqrt(jnp.float32(D))
        for h in range(NHEADS):
            qh = Q[:, h * dh:(h + 1) * dh]
            kh = kd[:, h * dh:(h + 1) * dh]
            vh = vd[:, h * dh:(h + 1) * dh]
            lg = lax.dot_general(qh, kh,
                                 (((1,), (1,)), ((), ()))) * scale
            lg = jnp.where(occ, lg, -1e30)
            A = jax.nn.softmax(lg, axis=-1)
            outs.append(qh + jnp.dot(A, vh))
        O = jnp.concatenate(outs, axis=1)
        O2 = O + jax.nn.relu(jnp.dot(O, wo_ref[...]) + bo_ref[...])
        gv = jnp.dot(wr_ref[...], O2) + br_ref[...]
        h1 = jax.nn.relu(jnp.dot(gv, wl1_ref[...]) + bl1_ref[...])
        lg2 = jnp.dot(h1, wl2_ref[...]) + bl2_ref[...]
        logp_ref[0] = jax.nn.log_softmax(lg2, axis=-1)
        gv_ref[0] = gv

    g1 = lambda i: (i, 0, 0)
    return pl.pallas_call(
        body, grid=(G,),
        in_specs=[pl.BlockSpec((1, R, D), g1),
                  pl.BlockSpec((1, R, 1), g1),
                  pl.BlockSpec((1, M, 2 * D), g1),
                  pl.BlockSpec((1, M, 1), g1),
                  pl.BlockSpec((1, 1, M), g1),
                  pl.BlockSpec((D, D), _full),
                  pl.BlockSpec((D,), _full1),
                  pl.BlockSpec((D, D), _full),
                  pl.BlockSpec((D,), _full1),
                  pl.BlockSpec((1, R), _full),
                  pl.BlockSpec((1,), _full1),
                  pl.BlockSpec((D, D), _full),
                  pl.BlockSpec((D,), _full1),
                  pl.BlockSpec((D, NCLS), _full),
                  pl.BlockSpec((NCLS,), _full1)],
        out_specs=[pl.BlockSpec((1, 1, NCLS), g1),
                   pl.BlockSpec((1, 1, D), g1)],
        out_shape=[jax.ShapeDtypeStruct((G, 1, NCLS), jnp.float32),
                   jax.ShapeDtypeStruct((G, 1, D), jnp.float32)],
    )(xrows, vals, kvd, mask_m1, mask_1m, Wq, bq, Wo, bo, wread, bread,
      Wl1, bl1, Wl2, bl2)


def kernel(x, edge_index, batch, params):
    p = params
    src, dst = edge_index[0], edge_index[1]

    k_e = _ceil_div(E, NW * CH)
    src3 = _pad_to_tiles(src, 0, k_e)
    dst3 = _pad_to_tiles(dst, DUMP, k_e)

    k_c = _ceil_div(E + N, NW * CH)
    cnt_dst = jnp.concatenate(
        [dst.astype(jnp.int32), batch.astype(jnp.int32) + CNT_BASE])
    cnt3 = _pad_to_tiles(cnt_dst, DUMP_CNT, k_c)
    ones16 = jnp.ones((CH, 16), jnp.float32)
    zeros16 = jnp.zeros((NPAD // NS, 16), jnp.float32)
    cnt = _sc_counts(cnt3, ones16, zeros16, k=k_c)

    zeros128 = jnp.zeros((NPAD // NS, D), jnp.float32)

    def prop(t):
        return _sc_prop(t, src3, dst3, zeros128, d=D, k=k_e)

    def propn(tcol):
        return _sc_prop_narrow(tcol[:, 0], src3, dst3, zeros16, k=k_e)

    ta, tb = _tc_prep(x, cnt, p["W_enc"], p["b_enc"],
                      p["W_g1_0"], p["W_g1_1"])
    x1, ta2, tb2, ts1, sa1 = _tc_layer(
        prop(ta), prop(tb), ta, tb, cnt, p["b_g1_0"], p["b_g1_1"],
        p["W_g2_0"], p["W_g2_1"], p["W_ws2"], p["W_ws1"], p["b_ws1"],
        has_next=True)
    zt1 = propn(ts1)
    x2, ta3, tb3, ts2, sa2 = _tc_layer(
        prop(ta2), prop(tb2), ta2, tb2, cnt, p["b_g2_0"], p["b_g2_1"],
        p["W_g3_0"], p["W_g3_1"], p["W_ws2"], p["W_ws1"], p["b_ws1"],
        has_next=True)
    zt2 = propn(ts2)
    x3, ts3, sa3 = _tc_layer(
        prop(ta3), prop(tb3), ta3, tb3, cnt, p["b_g3_0"], p["b_g3_1"],
        p["W_g3_0"], p["W_g3_1"], p["W_ws2"], p["W_ws1"], p["b_ws1"],
        has_next=False)
    zt3 = propn(ts3)

    xm, tk, tv, tps, sap = _tc_combine(
        x1, x2, x3, sa1, sa2, sa3, ts1, ts2, ts3, zt1, zt2, zt3,
        cnt, p["b_ws2"], p["W_k"], p["W_v"], p["W_ps2"], p["W_ps1"],
        p["b_ps1"])
    kv, s_col = _tc_pool(prop(tk), prop(tv), propn(tps), tk, tv, tps, sap,
                         cnt, p["b_k"], p["b_v"], p["b_ps2"])

    didx, maskf = _tc_batchpos(cnt)
    k_g = _ceil_div(G * M, NW * CH)
    gidx3 = _pad_to_tiles(didx.reshape(-1), 0, k_g)
    kvd = _sc_gather(kv, gidx3, d=2 * D, k=k_g)[:G * M].reshape(G, M, 2 * D)
    sd = _sc_gather_narrow(s_col[:, 0], gidx3, k=k_g)[:G * M].reshape(G, M)

    vals, sel = _tc_topk(sd, maskf, didx)

    k_x = _ceil_div(G * R, NW * CH)
    sidx3 = _pad_to_tiles(sel.reshape(-1), 0, k_x)
    xrows = _sc_gather(xm, sidx3, d=D, k=k_x)[:G * R].reshape(G, R, D)
    logp3, gv3 = _tc_attn(
        xrows, vals.reshape(G, R, 1), kvd,
        maskf.reshape(G, M, 1), maskf.reshape(G, 1, M),
        p["W_q"], p["b_q"], p["W_o"], p["b_o"],
        p["w_read"].reshape(1, R), p["b_read"],
        p["W_l1"], p["b_l1"], p["W_l2"], p["b_l2"])
    return logp3.reshape(G, NCLS), gv3.reshape(G, D)


# speedup vs baseline: 1.0000x; 1.0000x over previous
import functools

import jax
import jax.numpy as jnp
from jax import lax
from jax.experimental import pallas as pl
from jax.experimental.pallas import tpu as pltpu
from jax.experimental.pallas import tpu_sc as plsc

N = 10000
E = 320000
G = 100
D = 128
NHEADS = 4
ALPHA = 0.5
R = 25
NCLS = 10
M = 160

NC, NS = 2, 16
NW = NC * NS
CH = 128
NPAD = 10240
DUMP = 10220
CNT_BASE = 10000
DUMP_CNT = 10230

_MESH = dict(core_axis_name="c", subcore_axis_name="s", num_cores=NC,
             num_subcores=NS)


def _ceil_div(a, b):
    return -(-a // b)


@functools.partial(jax.jit, static_argnames=("d", "k", "ch"))
def _sc_prop(table, src3, dst3, zeros, *, d, k, ch):
    rows_pt = NPAD // NS
    mesh = plsc.VectorSubcoreMesh(**_MESH)

    @functools.partial(
        pl.kernel,
        out_type=jax.ShapeDtypeStruct((NC, NPAD, d), jnp.float32),
        mesh=mesh,
        scratch_types=[
            pltpu.VMEM((k, ch), jnp.int32),
            pltpu.VMEM((k, ch), jnp.int32),
            pltpu.VMEM((ch, d), jnp.float32),
            pltpu.VMEM_SHARED((NPAD, d), jnp.float32),
            pltpu.SemaphoreType.DMA,
        ],
    )
    def kfn(table_hbm, src_hbm, dst_hbm, zero_hbm, out_hbm,
            sidx, didx, buf, accum, sem):
        c = lax.axis_index("c")
        s = lax.axis_index("s")
        w = c * NS + s
        pltpu.sync_copy(zero_hbm, accum.at[pl.ds(s * rows_pt, rows_pt)])
        pltpu.sync_copy(src_hbm.at[w], sidx)
        pltpu.sync_copy(dst_hbm.at[w], didx)
        plsc.subcore_barrier()

        def body(j, carry):
            pltpu.async_copy(table_hbm.at[sidx.at[j]], buf, sem).wait()
            pltpu.sync_copy(buf, accum.at[didx.at[j]], add=True)
            return carry

        lax.fori_loop(0, k, body, 0)
        plsc.subcore_barrier()
        pltpu.sync_copy(accum.at[pl.ds(s * rows_pt, rows_pt)],
                        out_hbm.at[c, pl.ds(s * rows_pt, rows_pt)])

    return kfn(table, src3, dst3, zeros)


@functools.partial(jax.jit, static_argnames=("k",))
def _sc_counts(dst3, ones, zeros, *, k):
    rows_pt = NPAD // NS
    mesh = plsc.VectorSubcoreMesh(**_MESH)

    @functools.partial(
        pl.kernel,
        out_type=jax.ShapeDtypeStruct((NC, NPAD, 16), jnp.float32),
        mesh=mesh,
        scratch_types=[
            pltpu.VMEM((k, CH), jnp.int32),
            pltpu.VMEM((CH, 16), jnp.float32),
            pltpu.VMEM_SHARED((NPAD, 16), jnp.float32),
        ],
        compiler_params=pltpu.CompilerParams(use_tc_tiling_on_sc=False),
    )
    def kfn(dst_hbm, ones_hbm, zero_hbm, out_hbm, didx, buf, accum):
        c = lax.axis_index("c")
        s = lax.axis_index("s")
        w = c * NS + s
        pltpu.sync_copy(zero_hbm, accum.at[pl.ds(s * rows_pt, rows_pt)])
        pltpu.sync_copy(dst_hbm.at[w], didx)
        pltpu.sync_copy(ones_hbm, buf)
        plsc.subcore_barrier()

        def body(j, carry):
            pltpu.sync_copy(buf, accum.at[didx.at[j]], add=True)
            return carry

        lax.fori_loop(0, k, body, 0)
        plsc.subcore_barrier()
        pltpu.sync_copy(accum.at[pl.ds(s * rows_pt, rows_pt)],
                        out_hbm.at[c, pl.ds(s * rows_pt, rows_pt)])

    return kfn(dst3, ones, zeros)


@functools.partial(jax.jit, static_argnames=("k",))
def _sc_prop_narrow(t, src3, dst3, zeros, *, k):
    rows_pt = NPAD // NS
    mesh = plsc.VectorSubcoreMesh(**_MESH)

    @functools.partial(
        pl.kernel,
        out_type=jax.ShapeDtypeStruct((NC, NPAD, 16), jnp.float32),
        mesh=mesh,
        scratch_types=[
            pltpu.VMEM((N,), jnp.float32),
            pltpu.VMEM((k, CH), jnp.int32),
            pltpu.VMEM((k, CH), jnp.int32),
            pltpu.VMEM((CH, 16), jnp.float32),
            pltpu.VMEM_SHARED((NPAD, 16), jnp.float32),
        ],
        compiler_params=pltpu.CompilerParams(
            needs_layout_passes=False, use_tc_tiling_on_sc=False),
    )
    def kfn(t_hbm, src_hbm, dst_hbm, zero_hbm, out_hbm,
            tv, sidx, didx, buf, accum):
        c = lax.axis_index("c")
        s = lax.axis_index("s")
        w = c * NS + s
        pltpu.sync_copy(zero_hbm, accum.at[pl.ds(s * rows_pt, rows_pt)])
        pltpu.sync_copy(t_hbm, tv)
        pltpu.sync_copy(src_hbm.at[w], sidx)
        pltpu.sync_copy(dst_hbm.at[w], didx)
        pltpu.sync_copy(zero_hbm.at[pl.ds(0, CH)], buf)
        plsc.subcore_barrier()
        lane = lax.iota(jnp.int32, 16)
        col0 = jnp.zeros((16,), jnp.int32)

        def chunk(j, carry):
            def grp(g, c2):
                sv = sidx[j, pl.ds(g * 16, 16)]
                vals = plsc.load_gather(tv, [sv])
                plsc.store_scatter(buf, [g * 16 + lane, col0], vals)
                return c2

            lax.fori_loop(0, 8, grp, 0)
            pltpu.sync_copy(buf, accum.at[didx.at[j]], add=True)
            return carry

        lax.fori_loop(0, k, chunk, 0)
        plsc.subcore_barrier()
        pltpu.sync_copy(accum.at[pl.ds(s * rows_pt, rows_pt)],
                        out_hbm.at[c, pl.ds(s * rows_pt, rows_pt)])

    return kfn(t, src3, dst3, zeros)


@functools.partial(jax.jit, static_argnames=("k",))
def _sc_gather_narrow(t, idx3, *, k):
    mesh = plsc.VectorSubcoreMesh(**_MESH)

    @functools.partial(
        pl.kernel,
        out_type=jax.ShapeDtypeStruct((NW * k * CH,), jnp.float32),
        mesh=mesh,
        scratch_types=[
            pltpu.VMEM((N,), jnp.float32),
            pltpu.VMEM((k, CH), jnp.int32),
            pltpu.VMEM((k * CH,), jnp.float32),
        ],
        compiler_params=pltpu.CompilerParams(needs_layout_passes=False),
    )
    def kfn(t_hbm, idx_hbm, out_hbm, tv, idxb, obuf):
        c = lax.axis_index("c")
        s = lax.axis_index("s")
        w = c * NS + s
        pltpu.sync_copy(t_hbm, tv)
        pltpu.sync_copy(idx_hbm.at[w], idxb)

        def chunk(j, carry):
            def grp(g, c2):
                sv = idxb[j, pl.ds(g * 16, 16)]
                obuf[pl.ds(j * CH + g * 16, 16)] = plsc.load_gather(tv, [sv])
                return c2

            lax.fori_loop(0, 8, grp, 0)
            return carry

        lax.fori_loop(0, k, chunk, 0)
        pltpu.sync_copy(obuf, out_hbm.at[pl.ds(w * k * CH, k * CH)])

    return kfn(t, idx3)


@functools.partial(jax.jit, static_argnames=("d", "k"))
def _sc_gather(table, idx3, *, d, k):
    mesh = plsc.VectorSubcoreMesh(**_MESH)

    @functools.partial(
        pl.kernel,
        out_type=jax.ShapeDtypeStruct((NW * k * CH, d), jnp.float32),
        mesh=mesh,
        scratch_types=[
            pltpu.VMEM((k, CH), jnp.int32),
            pltpu.VMEM((CH, d), jnp.float32),
            pltpu.SemaphoreType.DMA,
        ],
    )
    def kfn(table_hbm, idx_hbm, out_hbm, idxb, buf, sem):
        c = lax.axis_index("c")
        s = lax.axis_index("s")
        w = c * NS + s
        pltpu.sync_copy(idx_hbm.at[w], idxb)

        def body(j, carry):
            pltpu.async_copy(table_hbm.at[idxb.at[j]], buf, sem).wait()
            pltpu.sync_copy(buf, out_hbm.at[pl.ds(w * k * CH + j * CH, CH)])
            return carry

        lax.fori_loop(0, k, body, 0)

    return kfn(table, idx3)


def _pad_to_tiles(v, fill, k, ch=CH):
    tot = NW * k * ch
    v = jnp.concatenate(
        [v.astype(jnp.int32),
         jnp.full((tot - v.shape[0],), fill, jnp.int32)])
    return v.reshape(NW, k, ch)


BR = 1000
GN = N // BR

_rows = lambda i: (i, 0)
_rows3 = lambda i: (0, i, 0)
_full = lambda i: (0, 0)
_full1 = lambda i: (0,)


def _dinv_of(cnt_ref):
    deg = cnt_ref[0][:, 0:1] + cnt_ref[1][:, 0:1] + 1.0
    return lax.rsqrt(jnp.maximum(deg, 1e-12))


@jax.jit
def _tc_prep(x, cnt, We, be, Wa, Wb):
    def body(x_ref, cnt_ref, we_ref, be_ref, wa_ref, wb_ref, ta_ref, tb_ref):
        dinv = _dinv_of(cnt_ref)
        h = jnp.dot(x_ref[...], we_ref[...]) + be_ref[...]
        ta_ref[...] = dinv * jnp.dot(h, wa_ref[...])
        tb_ref[...] = dinv * jnp.dot(h, wb_ref[...])

    return pl.pallas_call(
        body, grid=(GN,),
        in_specs=[pl.BlockSpec((BR, D), _rows),
                  pl.BlockSpec((NC, BR, 16), _rows3),
                  pl.BlockSpec((D, D), _full),
                  pl.BlockSpec((D,), _full1),
                  pl.BlockSpec((D, D), _full),
                  pl.BlockSpec((D, D), _full)],
        out_specs=[pl.BlockSpec((BR, D), _rows)] * 2,
        out_shape=[jax.ShapeDtypeStruct((N, D), jnp.float32)] * 2,
    )(x, cnt, We, be, Wa, Wb)


@functools.partial(jax.jit, static_argnames=("has_next",))
def _tc_layer(za, zb, tsa, tsb, cnt, ba, bb, Wna, Wnb, Wws2, Wws1, bws1,
              *, has_next):
    def body(za_ref, zb_ref, tsa_ref, tsb_ref, cnt_ref, ba_ref, bb_ref,
             wna_ref, wnb_ref, wws2_ref, wws1_ref, bws1_ref, *out_refs):
        if has_next:
            x_ref, tna_ref, tnb_ref, ts_ref, sa_ref = out_refs
        else:
            x_ref, ts_ref, sa_ref = out_refs
        dinv = _dinv_of(cnt_ref)
        conv_a = dinv * (za_ref[0] + za_ref[1] + tsa_ref[...]) + ba_ref[...]
        conv_b = dinv * (zb_ref[0] + zb_ref[1] + tsb_ref[...]) + bb_ref[...]
        xi = jax.nn.relu(conv_a) + jax.nn.relu(conv_b)
        x_ref[...] = xi
        if has_next:
            tna_ref[...] = dinv * jnp.dot(xi, wna_ref[...])
            tnb_ref[...] = dinv * jnp.dot(xi, wnb_ref[...])
        ts_ref[...] = dinv * jnp.dot(xi, wws2_ref[...])
        sa_ref[...] = jnp.dot(xi, wws1_ref[...]) + bws1_ref[...]

    n_out = 5 if has_next else 3
    shapes = [jax.ShapeDtypeStruct((N, D), jnp.float32)] * (3 if has_next else 1) \
        + [jax.ShapeDtypeStruct((N, 1), jnp.float32)] * 2
    specs = [pl.BlockSpec((BR, D), _rows)] * (3 if has_next else 1) \
        + [pl.BlockSpec((BR, 1), _rows)] * 2
    assert len(shapes) == n_out
    return pl.pallas_call(
        body, grid=(GN,),
        in_specs=[pl.BlockSpec((NC, BR, D), _rows3),
                  pl.BlockSpec((NC, BR, D), _rows3),
                  pl.BlockSpec((BR, D), _rows),
                  pl.BlockSpec((BR, D), _rows),
                  pl.BlockSpec((NC, BR, 16), _rows3),
                  pl.BlockSpec((D,), _full1),
                  pl.BlockSpec((D,), _full1),
                  pl.BlockSpec((D, D), _full),
                  pl.BlockSpec((D, D), _full),
                  pl.BlockSpec((D, 1), _full),
                  pl.BlockSpec((D, 1), _full),
                  pl.BlockSpec((1,), _full1)],
        out_specs=specs,
        out_shape=shapes,
    )(za, zb, tsa, tsb, cnt, ba, bb, Wna, Wnb, Wws2, Wws1, bws1)


@jax.jit
def _tc_combine(x1, x2, x3, sa1, sa2, sa3, ts1, ts2, ts3, zt1, zt2, zt3,
                cnt, bws2, Wk, Wv, Wps2, Wps1, bps1):
    def body(x1_ref, x2_ref, x3_ref, sa1_ref, sa2_ref, sa3_ref,
             ts1_ref, ts2_ref, ts3_ref, zt1_ref, zt2_ref, zt3_ref,
             cnt_ref, bws2_ref, wk_ref, wv_ref, wps2_ref, wps1_ref, bps1_ref,
             xm_ref, tk_ref, tv_ref, tps_ref, sap_ref):
        dinv = _dinv_of(cnt_ref)

        def wcol(sa_ref, ts_ref, zt_ref):
            convn = dinv * (zt_ref[0][:, 0:1] + zt_ref[1][:, 0:1]
                            + ts_ref[...]) + bws2_ref[...]
            return ALPHA * sa_ref[...] + (1 - ALPHA) * convn

        wcat = jnp.concatenate(
            [wcol(sa1_ref, ts1_ref, zt1_ref),
             wcol(sa2_ref, ts2_ref, zt2_ref),
             wcol(sa3_ref, ts3_ref, zt3_ref)], axis=1)
        wsm = jax.nn.softmax(wcat, axis=-1)
        xm = (wsm[:, 0:1] * x1_ref[...] + wsm[:, 1:2] * x2_ref[...]
              + wsm[:, 2:3] * x3_ref[...])
        xm_ref[...] = xm
        tk_ref[...] = dinv * jnp.dot(xm, wk_ref[...])
        tv_ref[...] = dinv * jnp.dot(xm, wv_ref[...])
        tps_ref[...] = dinv * jnp.dot(xm, wps2_ref[...])
        sap_ref[...] = jnp.dot(xm, wps1_ref[...]) + bps1_ref[...]

    return pl.pallas_call(
        body, grid=(GN,),
        in_specs=[pl.BlockSpec((BR, D), _rows)] * 3
        + [pl.BlockSpec((BR, 1), _rows)] * 6
        + [pl.BlockSpec((NC, BR, 16), _rows3)] * 4
        + [pl.BlockSpec((1,), _full1),
           pl.BlockSpec((D, D), _full), pl.BlockSpec((D, D), _full),
           pl.BlockSpec((D, 1), _full), pl.BlockSpec((D, 1), _full),
           pl.BlockSpec((1,), _full1)],
        out_specs=[pl.BlockSpec((BR, D), _rows)] * 3
        + [pl.BlockSpec((BR, 1), _rows)] * 2,
        out_shape=[jax.ShapeDtypeStruct((N, D), jnp.float32)] * 3
        + [jax.ShapeDtypeStruct((N, 1), jnp.float32)] * 2,
    )(x1, x2, x3, sa1, sa2, sa3, ts1, ts2, ts3, zt1, zt2, zt3,
      cnt, bws2, Wk, Wv, Wps2, Wps1, bps1)


@jax.jit
def _tc_pool(zk, zv, zp, tk, tv, tps, sap, cnt, bk, bv, bps2):
    def body(zk_ref, zv_ref, zp_ref, tk_ref, tv_ref, tps_ref, sap_ref,
             cnt_ref, bk_ref, bv_ref, bps2_ref, kv_ref, s_ref):
        dinv = _dinv_of(cnt_ref)
        kv_ref[:, 0:D] = dinv * (zk_ref[0] + zk_ref[1] + tk_ref[...]) \
            + bk_ref[...]
        kv_ref[:, D:2 * D] = dinv * (zv_ref[0] + zv_ref[1] + tv_ref[...]) \
            + bv_ref[...]
        convp = dinv * (zp_ref[0][:, 0:1] + zp_ref[1][:, 0:1]
                        + tps_ref[...]) + bps2_ref[...]
        s_ref[...] = ALPHA * sap_ref[...] + (1 - ALPHA) * convp

    return pl.pallas_call(
        body, grid=(GN,),
        in_specs=[pl.BlockSpec((NC, BR, D), _rows3)] * 2
        + [pl.BlockSpec((NC, BR, 16), _rows3)]
        + [pl.BlockSpec((BR, D), _rows)] * 2
        + [pl.BlockSpec((BR, 1), _rows)] * 2
        + [pl.BlockSpec((NC, BR, 16), _rows3)]
        + [pl.BlockSpec((D,), _full1)] * 2
        + [pl.BlockSpec((1,), _full1)],
        out_specs=[pl.BlockSpec((BR, 2 * D), _rows),
                   pl.BlockSpec((BR, 1), _rows)],
        out_shape=[jax.ShapeDtypeStruct((N, 2 * D), jnp.float32),
                   jax.ShapeDtypeStruct((N, 1), jnp.float32)],
    )(zk, zv, zp, tk, tv, tps, sap, cnt, bk, bv, bps2)


@jax.jit
def _tc_batchpos(cnt):
    def body(cnt_ref, didx_ref, mask_ref):
        counts = (cnt_ref[0][0:G, 0:1] + cnt_ref[1][0:G, 0:1])
        row = lax.broadcasted_iota(jnp.int32, (G, G), 0)
        col = lax.broadcasted_iota(jnp.int32, (G, G), 1)
        tri = (col < row).astype(jnp.float32)
        starts = jax.lax.dot_general(
            tri, counts, (((1,), (0,)), ((), ())),
            precision=jax.lax.Precision.HIGHEST)
        midx = lax.broadcasted_iota(jnp.int32, (G, M), 1)
        didx = jnp.clip(starts.astype(jnp.int32) + midx, 0, N - 1)
        didx_ref[...] = didx
        mask_ref[...] = (midx < counts.astype(jnp.int32)).astype(jnp.float32)

    return pl.pallas_call(
        body, grid=(1,),
        in_specs=[pl.BlockSpec((NC, 200, 16),
                               lambda i: (0, CNT_BASE // 200, 0))],
        out_specs=[pl.BlockSpec((G, M), _rows)] * 2,
        out_shape=[jax.ShapeDtypeStruct((G, M), jnp.int32),
                   jax.ShapeDtypeStruct((G, M), jnp.float32)],
    )(cnt)


@jax.jit
def _tc_topk(sd, maskf, didx):
    def body(sd_ref, mask_ref, didx_ref, vals_ref, sel_ref):
        occ = mask_ref[...] > 0.0
        cur = jnp.where(occ, sd_ref[...], -1e30)
        di = jnp.where(occ, didx_ref[...], 0)
        iot = lax.broadcasted_iota(jnp.int32, (G, M), 1)
        big = jnp.int32(1 << 30)
        for r in range(R):
            mx = jnp.max(cur, axis=1, keepdims=True)
            ismax = cur == mx
            am = jnp.min(jnp.where(ismax, iot, big), axis=1, keepdims=True)
            take = iot == am
            vals_ref[:, r:r + 1] = mx
            sel_ref[:, r:r + 1] = jnp.max(jnp.where(take, di, 0), axis=1,
                                          keepdims=True)
            cur = jnp.where(take, jnp.float32(-3e38), cur)

    return pl.pallas_call(
        body, grid=(1,),
        in_specs=[pl.BlockSpec((G, M), _rows)] * 3,
        out_specs=[pl.BlockSpec((G, R), _rows)] * 2,
        out_shape=[jax.ShapeDtypeStruct((G, R), jnp.float32),
                   jax.ShapeDtypeStruct((G, R), jnp.int32)],
    )(sd, maskf, didx)


@jax.jit
def _tc_attn(xrows, vals, kvd, mask_m1, mask_1m, Wq, bq, Wo, bo,
             wread, bread, Wl1, bl1, Wl2, bl2):
    dh = D // NHEADS

    def body(xr_ref, vals_ref, kvd_ref, mm1_ref, m1m_ref, wq_ref, bq_ref,
             wo_ref, bo_ref, wr_ref, br_ref, wl1_ref, bl1_ref,
             wl2_ref, bl2_ref, logp_ref, gv_ref):
        v = vals_ref[0]
        ok = v > -1e29
        xp = jnp.where(ok, xr_ref[0] * jnp.tanh(v), 0.0)
        Q = jnp.dot(xp, wq_ref[...]) + bq_ref[...]
        mcol = mm1_ref[0]
        occ = m1m_ref[0] > 0.0
        kd = kvd_ref[0][:, 0:D] * mcol
        vd = kvd_ref[0][:, D:2 * D] * mcol
        outs = []
        scale = 1.0 / jnp.sqrt(jnp.float32(D))
        for h in range(NHEADS):
            qh = Q[:, h * dh:(h + 1) * dh]
            kh = kd[:, h * dh:(h + 1) * dh]
            vh = vd[:, h * dh:(h + 1) * dh]
            lg = lax.dot_general(qh, kh,
                                 (((1,), (1,)), ((), ()))) * scale
            lg = jnp.where(occ, lg, -1e30)
            A = jax.nn.softmax(lg, axis=-1)
            outs.append(qh + jnp.dot(A, vh))
        O = jnp.concatenate(outs, axis=1)
        O2 = O + jax.nn.relu(jnp.dot(O, wo_ref[...]) + bo_ref[...])
        gv = jnp.dot(wr_ref[...], O2) + br_ref[...]
        h1 = jax.nn.relu(jnp.dot(gv, wl1_ref[...]) + bl1_ref[...])
        lg2 = jnp.dot(h1, wl2_ref[...]) + bl2_ref[...]
        logp_ref[0] = jax.nn.log_softmax(lg2, axis=-1)
        gv_ref[0] = gv

    g1 = lambda i: (i, 0, 0)
    return pl.pallas_call(
        body, grid=(G,),
        in_specs=[pl.BlockSpec((1, R, D), g1),
                  pl.BlockSpec((1, R, 1), g1),
                  pl.BlockSpec((1, M, 2 * D), g1),
                  pl.BlockSpec((1, M, 1), g1),
                  pl.BlockSpec((1, 1, M), g1),
                  pl.BlockSpec((D, D), _full),
                  pl.BlockSpec((D,), _full1),
                  pl.BlockSpec((D, D), _full),
                  pl.BlockSpec((D,), _full1),
                  pl.BlockSpec((1, R), _full),
                  pl.BlockSpec((1,), _full1),
                  pl.BlockSpec((D, D), _full),
                  pl.BlockSpec((D,), _full1),
                  pl.BlockSpec((D, NCLS), _full),
                  pl.BlockSpec((NCLS,), _full1)],
        out_specs=[pl.BlockSpec((1, 1, NCLS), g1),
                   pl.BlockSpec((1, 1, D), g1)],
        out_shape=[jax.ShapeDtypeStruct((G, 1, NCLS), jnp.float32),
                   jax.ShapeDtypeStruct((G, 1, D), jnp.float32)],
    )(xrows, vals, kvd, mask_m1, mask_1m, Wq, bq, Wo, bo, wread, bread,
      Wl1, bl1, Wl2, bl2)


def kernel(x, edge_index, batch, params):
    p = params
    src, dst = edge_index[0], edge_index[1]

    k_e = _ceil_div(E, NW * CH)
    src3 = _pad_to_tiles(src, 0, k_e)
    dst3 = _pad_to_tiles(dst, DUMP, k_e)

    k_c = _ceil_div(E + N, NW * CH)
    cnt_dst = jnp.concatenate(
        [dst.astype(jnp.int32), batch.astype(jnp.int32) + CNT_BASE])
    cnt3 = _pad_to_tiles(cnt_dst, DUMP_CNT, k_c)
    ones16 = jnp.ones((CH, 16), jnp.float32)
    zeros16 = jnp.zeros((NPAD // NS, 16), jnp.float32)
    cnt = _sc_counts(cnt3, ones16, zeros16, k=k_c)

    zeros128 = jnp.zeros((NPAD // NS, D), jnp.float32)

    def prop(t):
        return _sc_prop(t, src3, dst3, zeros128, d=D, k=k_e, ch=CH)

    def propn(tcol):
        return _sc_prop_narrow(tcol[:, 0], src3, dst3, zeros16, k=k_e)

    ta, tb = _tc_prep(x, cnt, p["W_enc"], p["b_enc"],
                      p["W_g1_0"], p["W_g1_1"])
    x1, ta2, tb2, ts1, sa1 = _tc_layer(
        prop(ta), prop(tb), ta, tb, cnt, p["b_g1_0"], p["b_g1_1"],
        p["W_g2_0"], p["W_g2_1"], p["W_ws2"], p["W_ws1"], p["b_ws1"],
        has_next=True)
    zt1 = propn(ts1)
    x2, ta3, tb3, ts2, sa2 = _tc_layer(
        prop(ta2), prop(tb2), ta2, tb2, cnt, p["b_g2_0"], p["b_g2_1"],
        p["W_g3_0"], p["W_g3_1"], p["W_ws2"], p["W_ws1"], p["b_ws1"],
        has_next=True)
    zt2 = propn(ts2)
    x3, ts3, sa3 = _tc_layer(
        prop(ta3), prop(tb3), ta3, tb3, cnt, p["b_g3_0"], p["b_g3_1"],
        p["W_g3_0"], p["W_g3_1"], p["W_ws2"], p["W_ws1"], p["b_ws1"],
        has_next=False)
    zt3 = propn(ts3)

    xm, tk, tv, tps, sap = _tc_combine(
        x1, x2, x3, sa1, sa2, sa3, ts1, ts2, ts3, zt1, zt2, zt3,
        cnt, p["b_ws2"], p["W_k"], p["W_v"], p["W_ps2"], p["W_ps1"],
        p["b_ps1"])
    kv, s_col = _tc_pool(prop(tk), prop(tv), propn(tps), tk, tv, tps, sap,
                         cnt, p["b_k"], p["b_v"], p["b_ps2"])

    didx, maskf = _tc_batchpos(cnt)
    k_g = _ceil_div(G * M, NW * CH)
    gidx3 = _pad_to_tiles(didx.reshape(-1), 0, k_g)
    kvd = _sc_gather(kv, gidx3, d=2 * D, k=k_g)[:G * M].reshape(G, M, 2 * D)
    sd = _sc_gather_narrow(s_col[:, 0], gidx3, k=k_g)[:G * M].reshape(G, M)

    vals, sel = _tc_topk(sd, maskf, didx)

    k_x = _ceil_div(G * R, NW * CH)
    sidx3 = _pad_to_tiles(sel.reshape(-1), 0, k_x)
    xrows = _sc_gather(xm, sidx3, d=D, k=k_x)[:G * R].reshape(G, R, D)
    logp3, gv3 = _tc_attn(
        xrows, vals.reshape(G, R, 1), kvd,
        maskf.reshape(G, M, 1), maskf.reshape(G, 1, M),
        p["W_q"], p["b_q"], p["W_o"], p["b_o"],
        p["w_read"].reshape(1, R), p["b_read"],
        p["W_l1"], p["b_l1"], p["W_l2"], p["b_l2"])
    return logp3.reshape(G, NCLS), gv3.reshape(G, D)
